# baseline (speedup 1.0000x reference)
import functools

import jax
import jax.numpy as jnp
from jax import lax
from jax.experimental import pallas as pl
from jax.experimental.pallas import tpu as pltpu
from jax.experimental.pallas import tpu_sc as plsc

NN = 100000
NSEG = 1000
OB = 1008
NC, NS = 2, 16
CH = 6400
PASS = 3200
NV = PASS // 16
SPW = 64
INF = float("inf")
IMAX = 2147483647

_mesh = plsc.VectorSubcoreMesh(
    core_axis_name="c", subcore_axis_name="s", num_cores=NC, num_subcores=NS)


def _iota16():
    return lax.broadcasted_iota(jnp.int32, (16,), 0)


def _bc(x):
    return jnp.broadcast_to(x, (16,))


@functools.partial(
    pl.kernel,
    out_type=(
        jax.ShapeDtypeStruct((NSEG,), jnp.float32),
        jax.ShapeDtypeStruct((NSEG * 3,), jnp.float32),
    ),
    mesh=_mesh,
    compiler_params=pltpu.CompilerParams(
        use_tc_tiling_on_sc=False, needs_layout_passes=False),
    scratch_types=[
        pltpu.VMEM((PASS, 16), jnp.float32),
        pltpu.VMEM((PASS, 16), jnp.float32),
        pltpu.VMEM((CH,), jnp.float32),
        pltpu.VMEM((CH,), jnp.int32),
        pltpu.VMEM((OB,), jnp.float32),
        pltpu.VMEM((OB,), jnp.int32),
        pltpu.VMEM_SHARED((NS, OB), jnp.float32),
        pltpu.VMEM_SHARED((NS, OB), jnp.int32),
        pltpu.VMEM((NS, SPW), jnp.float32),
        pltpu.VMEM((NS, SPW), jnp.int32),
        pltpu.VMEM((SPW,), jnp.int32),
        pltpu.VMEM((2, SPW * 3 // 2), jnp.int32),
        pltpu.VMEM((SPW * 3,), jnp.float32),
        pltpu.VMEM((SPW,), jnp.float32),
        pltpu.SemaphoreType.DMA,
        pltpu.SemaphoreType.DMA,
        pltpu.SemaphoreType.DMA,
    ],
)
def _pick(h_hbm, chi_hbm, bidx_hbm, posf_hbm, ptr_hbm, pdir_hbm,
          h16a_v, h16b_v, chi_v, bidx_v, oval_v, oidx_v, shval, shidx,
          pv_v, pi_v, picks_v, gidx_v, pbuf_v, ptr_v, sema, semb, semg):
    core = lax.axis_index("c")

    @pl.when(core == 0)
    def _():
        t = lax.axis_index("s")
        base = jnp.minimum(t * CH, NN - CH)
        lanes = _iota16()
        lane0 = lanes == 0

        cpa = pltpu.async_copy(
            h_hbm.at[pl.ds(base, PASS), pl.ds(0, 16)], h16a_v, sema)
        cpb = pltpu.async_copy(
            h_hbm.at[pl.ds(base + PASS, PASS), pl.ds(0, 16)], h16b_v, semb)
        pltpu.sync_copy(chi_hbm.at[pl.ds(base, CH)], chi_v)
        pltpu.sync_copy(bidx_hbm.at[pl.ds(base, CH)], bidx_v)

        inf_vec = jnp.full((16,), INF, jnp.float32)
        big_idx = jnp.full((16,), NN, jnp.int32)

        def init(i, _):
            oval_v[pl.ds(i * 16, 16)] = inf_vec
            oidx_v[pl.ds(i * 16, 16)] = big_idx
            return 0

        lax.fori_loop(0, OB // 16, init, 0)

        def flush(cs, av, an):
            m = jnp.min(av)
            mvec = _bc(m)
            nodemin = jnp.min(jnp.where(av == mvec, an, NN))
            csv = _bc(cs)
            wm = lane0 & (mvec < INF)
            plsc.store_scatter(oval_v, [csv], mvec, mask=wm)
            plsc.store_scatter(oidx_v, [csv], _bc(nodemin), mask=wm)

        def make_step(hbuf, pbase):
            def step(j, carry):
                cs, av, an = carry
                off = j * 16
                vb = bidx_v[pl.ds(pbase + off, 16)]
                vc = chi_v[pl.ds(pbase + off, 16)]
                rows = _bc(off) + lanes
                h3 = plsc.load_gather(hbuf, [rows, _bc(jnp.int32(3))])
                h4 = plsc.load_gather(hbuf, [rows, _bc(jnp.int32(4))])
                h5 = plsc.load_gather(hbuf, [rows, _bc(jnp.int32(5))])
                h6 = plsc.load_gather(hbuf, [rows, _bc(jnp.int32(6))])
                filt = (h4 > h3) & (h4 >= h5) & (h4 >= h6)
                key = jnp.where(filt, vc, INF)
                node = _bc(base + pbase + off) + lanes
                vb0 = vb[0]
                vb15 = vb[15]

                def fast(carry):
                    cs, av, an = carry
                    upd = key < av
                    return cs, jnp.minimum(av, key), jnp.where(upd, node, an)

                def slow(carry):
                    cs, av, an = carry
                    csv = _bc(cs)
                    mc = vb == csv
                    upd = mc & (key < av)
                    av = jnp.where(upd, key, av)
                    an = jnp.where(upd, node, an)
                    flush(cs, av, an)

                    def cond(carry):
                        rem, cs, av, an = carry
                        return jnp.any(rem)

                    def body(carry):
                        rem, cs, av, an = carry
                        s = jnp.min(jnp.where(rem, vb, IMAX))
                        svec = _bc(s)
                        segm = vb == svec
                        kseg = jnp.where(segm, key, INF)
                        is_last = s == vb15

                        def mid(args):
                            cs, av, an = args
                            m = jnp.min(kseg)
                            mvec = _bc(m)
                            nodemin = jnp.min(
                                jnp.where(segm & (kseg == mvec), node, NN))
                            wm = lane0 & (mvec < INF)
                            plsc.store_scatter(oval_v, [svec], mvec, mask=wm)
                            plsc.store_scatter(
                                oidx_v, [svec], _bc(nodemin), mask=wm)
                            return cs, av, an

                        def last(args):
                            return s, kseg, node

                        cs, av, an = lax.cond(is_last, last, mid, (cs, av, an))
                        return rem & ~segm, cs, av, an

                    rem0 = ~mc
                    _, cs, av, an = lax.while_loop(
                        cond, body, (rem0, cs, av, an))
                    return cs, av, an

                is_fast = (vb0 == vb15) & (vb0 == cs)
                return lax.cond(is_fast, fast, slow, (cs, av, an))

            return step

        cs0 = bidx_v[pl.ds(0, 16)][0]
        carry = (cs0, jnp.full((16,), INF, jnp.float32),
                 jnp.zeros((16,), jnp.int32))
        cpa.wait()
        carry = lax.fori_loop(0, NV, make_step(h16a_v, 0), carry)
        cpb.wait()
        carry = lax.fori_loop(0, NV, make_step(h16b_v, PASS), carry)
        flush(*carry)

        pltpu.sync_copy(oval_v, shval.at[t])
        pltpu.sync_copy(oidx_v, shidx.at[t])
        plsc.subcore_barrier()

        seg0 = jnp.minimum(t * SPW, NSEG - SPW)
        pltpu.sync_copy(shval.at[:, pl.ds(seg0, SPW)], pv_v)
        pltpu.sync_copy(shidx.at[:, pl.ds(seg0, SPW)], pi_v)

        def pick_step(sl, _):
            colv = _bc(sl).astype(jnp.int32)
            a = plsc.load_gather(pv_v, [lanes, colv])
            m = jnp.min(a)
            mvec = _bc(m)
            tile = plsc.all_reduce_ffs(a == mvec)
            pk = plsc.load_gather(pi_v, [_bc(tile).astype(jnp.int32), colv])
            pickf = jnp.where(mvec < INF, pk, 0)
            plsc.store_scatter(picks_v, [colv], pickf, mask=lane0)
            return 0

        lax.fori_loop(0, SPW, pick_step, 0)

        def gidx_step(v, _):
            k = _bc(v * 16) + lanes
            p = k // 3
            c = k - p * 3
            pickp = plsc.load_gather(picks_v, [p])
            r = v // 6
            gidx_v[r, pl.ds((v - r * 6) * 16, 16)] = pickp * 3 + c
            return 0

        lax.fori_loop(0, (SPW * 3) // 16, gidx_step, 0)

        cg0 = pltpu.async_copy(
            posf_hbm.at[gidx_v.at[0]], pbuf_v.at[pl.ds(0, SPW * 3 // 2)], semg)
        cg1 = pltpu.async_copy(
            posf_hbm.at[gidx_v.at[1]],
            pbuf_v.at[pl.ds(SPW * 3 // 2, SPW * 3 // 2)], semg)
        cg0.wait()
        cg1.wait()

        def norm_step(hh, _):
            b3 = (_bc(hh * 16) + lanes) * 3
            x = plsc.load_gather(pbuf_v, [b3])
            y = plsc.load_gather(pbuf_v, [b3 + 1])
            z = plsc.load_gather(pbuf_v, [b3 + 2])
            s = x * x + y * y + z * z
            i = plsc.bitcast(s, jnp.int32)
            i = jnp.int32(0x1FBD1DF5) + (i >> 1)
            r = plsc.bitcast(i, jnp.float32)
            r = 0.5 * (r + s / r)
            r = 0.5 * (r + s / r)
            r = 0.5 * (r + s / r)
            r = jnp.where(s > 0.0, r, 0.0)
            ptr_v[pl.ds(hh * 16, 16)] = r
            return 0

        lax.fori_loop(0, SPW // 16, norm_step, 0)

        pltpu.sync_copy(ptr_v, ptr_hbm.at[pl.ds(seg0, SPW)])
        pltpu.sync_copy(pbuf_v, pdir_hbm.at[pl.ds(seg0 * 3, SPW * 3)])


def kernel(x_global_features, h, pos_pxpypz_at_vertex, chi_squared_tracks, batch_idx):
    del x_global_features
    posf = jnp.reshape(pos_pxpypz_at_vertex, (-1,))
    p_tracks, pdir_flat = _pick(
        h, chi_squared_tracks, batch_idx.astype(jnp.int32), posf)
    return p_tracks, jnp.reshape(pdir_flat, (NSEG, 3))


# speedup vs baseline: 1.6402x; 1.6402x over previous
import functools

import jax
import jax.numpy as jnp
from jax import lax
from jax.experimental import pallas as pl
from jax.experimental.pallas import tpu as pltpu
from jax.experimental.pallas import tpu_sc as plsc

NN = 100000
NSEG = 1000
OB = 1024
NC, NS = 2, 16
NW = NC * NS
CH = 3200
NV = CH // 16
SPW = 128
INF = float("inf")
IMAX = 2147483647

_mesh = plsc.VectorSubcoreMesh(
    core_axis_name="c", subcore_axis_name="s", num_cores=NC, num_subcores=NS)


def _iota16():
    return lax.broadcasted_iota(jnp.int32, (16,), 0)


def _bc(x):
    return jnp.broadcast_to(x, (16,))


@functools.partial(
    pl.kernel,
    out_type=(
        jax.ShapeDtypeStruct((NW, OB), jnp.float32),
        jax.ShapeDtypeStruct((NW, OB), jnp.int32),
    ),
    mesh=_mesh,
    compiler_params=pltpu.CompilerParams(
        use_tc_tiling_on_sc=False, needs_layout_passes=False),
    scratch_types=[
        pltpu.VMEM((CH, 16), jnp.float32),
        pltpu.VMEM((CH,), jnp.float32),
        pltpu.VMEM((CH,), jnp.int32),
        pltpu.VMEM((OB,), jnp.float32),
        pltpu.VMEM((OB,), jnp.int32),
        pltpu.SemaphoreType.DMA,
    ],
)
def _phase1(h_hbm, chi_hbm, bidx_hbm, pval_hbm, pidx_hbm,
            h16_v, chi_v, bidx_v, oval_v, oidx_v, sema):
    wid = lax.axis_index("s") * NC + lax.axis_index("c")
    base = jnp.minimum(wid * CH, NN - CH)
    lanes = _iota16()
    lane0 = lanes == 0

    cpa = pltpu.async_copy(
        h_hbm.at[pl.ds(base, CH), pl.ds(0, 16)], h16_v, sema)
    pltpu.sync_copy(chi_hbm.at[pl.ds(base, CH)], chi_v)
    pltpu.sync_copy(bidx_hbm.at[pl.ds(base, CH)], bidx_v)

    inf_vec = jnp.full((16,), INF, jnp.float32)
    big_idx = jnp.full((16,), NN, jnp.int32)

    def init(i, _):
        oval_v[pl.ds(i * 16, 16)] = inf_vec
        oidx_v[pl.ds(i * 16, 16)] = big_idx
        return 0

    lax.fori_loop(0, OB // 16, init, 0)

    def flush(cs, av, an):
        m = jnp.min(av)
        mvec = _bc(m)
        nodemin = jnp.min(jnp.where(av == mvec, an, NN))
        csv = _bc(cs)
        wm = lane0 & (mvec < INF)
        plsc.store_scatter(oval_v, [csv], mvec, mask=wm)
        plsc.store_scatter(oidx_v, [csv], _bc(nodemin), mask=wm)

    def step(j, carry):
        cs, av, an = carry
        off = j * 16
        vb = bidx_v[pl.ds(off, 16)]
        vc = chi_v[pl.ds(off, 16)]
        rows = _bc(off) + lanes
        h3 = plsc.load_gather(h16_v, [rows, _bc(jnp.int32(3))])
        h4 = plsc.load_gather(h16_v, [rows, _bc(jnp.int32(4))])
        h5 = plsc.load_gather(h16_v, [rows, _bc(jnp.int32(5))])
        h6 = plsc.load_gather(h16_v, [rows, _bc(jnp.int32(6))])
        filt = (h4 > h3) & (h4 >= h5) & (h4 >= h6)
        key = jnp.where(filt, vc, INF)
        node = _bc(base + off) + lanes
        vb0 = vb[0]
        vb15 = vb[15]

        def fast(carry):
            cs, av, an = carry
            upd = key < av
            return cs, jnp.minimum(av, key), jnp.where(upd, node, an)

        def slow(carry):
            cs, av, an = carry
            csv = _bc(cs)
            mc = vb == csv
            upd = mc & (key < av)
            av = jnp.where(upd, key, av)
            an = jnp.where(upd, node, an)
            flush(cs, av, an)

            def cond(carry):
                rem, cs, av, an = carry
                return jnp.any(rem)

            def body(carry):
                rem, cs, av, an = carry
                s = jnp.min(jnp.where(rem, vb, IMAX))
                svec = _bc(s)
                segm = vb == svec
                kseg = jnp.where(segm, key, INF)
                is_last = s == vb15

                def mid(args):
                    cs, av, an = args
                    m = jnp.min(kseg)
                    mvec = _bc(m)
                    nodemin = jnp.min(
                        jnp.where(segm & (kseg == mvec), node, NN))
                    wm = lane0 & (mvec < INF)
                    plsc.store_scatter(oval_v, [svec], mvec, mask=wm)
                    plsc.store_scatter(oidx_v, [svec], _bc(nodemin), mask=wm)
                    return cs, av, an

                def last(args):
                    return s, kseg, node

                cs, av, an = lax.cond(is_last, last, mid, (cs, av, an))
                return rem & ~segm, cs, av, an

            rem0 = ~mc
            _, cs, av, an = lax.while_loop(cond, body, (rem0, cs, av, an))
            return cs, av, an

        is_fast = (vb0 == vb15) & (vb0 == cs)
        return lax.cond(is_fast, fast, slow, (cs, av, an))

    cs0 = bidx_v[pl.ds(0, 16)][0]
    carry = (cs0, jnp.full((16,), INF, jnp.float32),
             jnp.zeros((16,), jnp.int32))
    cpa.wait()
    carry = lax.fori_loop(0, NV, step, carry)
    flush(*carry)

    pltpu.sync_copy(oval_v, pval_hbm.at[wid])
    pltpu.sync_copy(oidx_v, pidx_hbm.at[wid])


@functools.partial(
    pl.kernel,
    out_type=(
        jax.ShapeDtypeStruct((OB,), jnp.float32),
        jax.ShapeDtypeStruct((OB * 3,), jnp.float32),
    ),
    mesh=_mesh,
    compiler_params=pltpu.CompilerParams(
        use_tc_tiling_on_sc=True, needs_layout_passes=False),
    scratch_types=[
        pltpu.VMEM((NW, SPW), jnp.float32),
        pltpu.VMEM((NW, SPW), jnp.int32),
        pltpu.VMEM((SPW,), jnp.int32),
        pltpu.VMEM((16, 8, 3), jnp.float32),
        pltpu.VMEM((SPW * 3,), jnp.float32),
        pltpu.VMEM((SPW,), jnp.float32),
        pltpu.SemaphoreType.DMA,
    ],
)
def _phase2(pval_hbm, pidx_hbm, pos_hbm, ptr_hbm, pdir_hbm,
            pv_v, pi_v, picks_v, posblk_v, pdir_v, ptr_v, semg):
    wid = lax.axis_index("s") * NC + lax.axis_index("c")

    @pl.when(wid < 8)
    def _():
        t = wid
        seg0 = t * SPW
        lanes = _iota16()
        lane0 = lanes == 0

        pltpu.sync_copy(pval_hbm.at[:, pl.ds(seg0, SPW)], pv_v)
        pltpu.sync_copy(pidx_hbm.at[:, pl.ds(seg0, SPW)], pi_v)

        def group(g, _):
            def pick_step(k, _):
                colv = _bc(g * 16 + k).astype(jnp.int32)
                a = plsc.load_gather(pv_v, [lanes, colv])
                b = plsc.load_gather(pv_v, [lanes + 16, colv])
                m = jnp.minimum(jnp.min(a), jnp.min(b))
                mvec = _bc(m)
                eqa = a == mvec
                anya = jnp.any(eqa)
                la = plsc.all_reduce_ffs(eqa)
                lb = plsc.all_reduce_ffs(b == mvec)
                tile = jnp.where(
                    anya, _bc(la), _bc(lb) + 16).astype(jnp.int32)
                pk = plsc.load_gather(pi_v, [tile, colv])
                pickf = jnp.where(mvec < INF, pk, 0)
                plsc.store_scatter(picks_v, [colv], pickf, mask=lane0)
                r0 = (pickf[0] // 8) * 8
                pltpu.async_copy(
                    pos_hbm.at[pl.ds(r0, 8), :], posblk_v.at[k], semg)
                return 0

            lax.fori_loop(0, 16, pick_step, 0)

            def drain(i, _):
                pltpu.make_async_copy(
                    pos_hbm.at[pl.ds(0, 8), :], posblk_v.at[0], semg).wait()
                return 0

            lax.fori_loop(0, 16, drain, 0)

            iv = _bc(g * 16) + lanes
            pkv = plsc.load_gather(picks_v, [iv])
            rv = pkv - (pkv // 8) * 8
            x = plsc.load_gather(posblk_v, [lanes, rv, _bc(jnp.int32(0))])
            y = plsc.load_gather(posblk_v, [lanes, rv, _bc(jnp.int32(1))])
            z = plsc.load_gather(posblk_v, [lanes, rv, _bc(jnp.int32(2))])
            s = x * x + y * y + z * z
            i = plsc.bitcast(s, jnp.int32)
            i = jnp.int32(0x1FBD1DF5) + (i >> 1)
            r = plsc.bitcast(i, jnp.float32)
            r = 0.5 * (r + s / r)
            r = 0.5 * (r + s / r)
            r = 0.5 * (r + s / r)
            r = jnp.where(s > 0.0, r, 0.0)
            ptr_v[pl.ds(g * 16, 16)] = r
            kv = iv * 3
            plsc.store_scatter(pdir_v, [kv], x)
            plsc.store_scatter(pdir_v, [kv + 1], y)
            plsc.store_scatter(pdir_v, [kv + 2], z)
            return 0

        lax.fori_loop(0, SPW // 16, group, 0)

        pltpu.sync_copy(ptr_v, ptr_hbm.at[pl.ds(seg0, SPW)])
        pltpu.sync_copy(pdir_v, pdir_hbm.at[pl.ds(seg0 * 3, SPW * 3)])


def kernel(x_global_features, h, pos_pxpypz_at_vertex, chi_squared_tracks, batch_idx):
    del x_global_features
    pval, pidx = _phase1(h, chi_squared_tracks, batch_idx.astype(jnp.int32))
    p_tracks, pdir_flat = _phase2(pval, pidx, pos_pxpypz_at_vertex)
    return p_tracks[:NSEG], jnp.reshape(pdir_flat[:NSEG * 3], (NSEG, 3))


# speedup vs baseline: 1.8400x; 1.1218x over previous
import functools

import jax
import jax.numpy as jnp
from jax import lax
from jax.experimental import pallas as pl
from jax.experimental.pallas import tpu as pltpu
from jax.experimental.pallas import tpu_sc as plsc

NN = 100000
NSEG = 1000
OB = 1024
NC, NS = 2, 16
NW = NC * NS
CH = 3200
NV = CH // 16
SPW = 128
INF = float("inf")
IMAX = 2147483647

_mesh = plsc.VectorSubcoreMesh(
    core_axis_name="c", subcore_axis_name="s", num_cores=NC, num_subcores=NS)


def _iota16():
    return lax.broadcasted_iota(jnp.int32, (16,), 0)


def _bc(x):
    return jnp.broadcast_to(x, (16,))


@functools.partial(
    pl.kernel,
    out_type=(
        jax.ShapeDtypeStruct((NW, OB), jnp.float32),
        jax.ShapeDtypeStruct((NW, OB), jnp.int32),
    ),
    mesh=_mesh,
    compiler_params=pltpu.CompilerParams(
        use_tc_tiling_on_sc=False, needs_layout_passes=False),
    scratch_types=[
        pltpu.VMEM((CH, 16), jnp.float32),
        pltpu.VMEM((CH,), jnp.float32),
        pltpu.VMEM((CH,), jnp.int32),
        pltpu.VMEM((OB,), jnp.float32),
        pltpu.VMEM((OB,), jnp.int32),
        pltpu.SemaphoreType.DMA,
    ],
)
def _phase1(h_hbm, chi_hbm, bidx_hbm, pval_hbm, pidx_hbm,
            h16_v, chi_v, bidx_v, oval_v, oidx_v, sema):
    wid = lax.axis_index("s") * NC + lax.axis_index("c")
    base = jnp.minimum(wid * CH, NN - CH)
    lanes = _iota16()
    lane0 = lanes == 0

    cpa = pltpu.async_copy(
        h_hbm.at[pl.ds(base, CH), pl.ds(0, 16)], h16_v, sema)
    pltpu.sync_copy(chi_hbm.at[pl.ds(base, CH)], chi_v)
    pltpu.sync_copy(bidx_hbm.at[pl.ds(base, CH)], bidx_v)

    inf_vec = jnp.full((16,), INF, jnp.float32)
    big_idx = jnp.full((16,), NN, jnp.int32)

    def init(i, _):
        oval_v[pl.ds(i * 16, 16)] = inf_vec
        oidx_v[pl.ds(i * 16, 16)] = big_idx
        return 0

    lax.fori_loop(0, OB // 16, init, 0)

    def flush(cs, av, an):
        m = jnp.min(av)
        mvec = _bc(m)
        nodemin = jnp.min(jnp.where(av == mvec, an, NN))
        csv = _bc(cs)
        wm = lane0 & (mvec < INF)
        plsc.store_scatter(oval_v, [csv], mvec, mask=wm)
        plsc.store_scatter(oidx_v, [csv], _bc(nodemin), mask=wm)

    def step(j, carry):
        cs, av, an = carry
        off = j * 16
        vb = bidx_v[pl.ds(off, 16)]
        vc = chi_v[pl.ds(off, 16)]
        rows = _bc(off) + lanes
        h3 = plsc.load_gather(h16_v, [rows, _bc(jnp.int32(3))])
        h4 = plsc.load_gather(h16_v, [rows, _bc(jnp.int32(4))])
        h5 = plsc.load_gather(h16_v, [rows, _bc(jnp.int32(5))])
        h6 = plsc.load_gather(h16_v, [rows, _bc(jnp.int32(6))])
        filt = (h4 > h3) & (h4 >= h5) & (h4 >= h6)
        key = jnp.where(filt, vc, INF)
        node = _bc(base + off) + lanes
        vb0 = vb[0]
        vb15 = vb[15]

        def fast(carry):
            cs, av, an = carry
            upd = key < av
            return cs, jnp.minimum(av, key), jnp.where(upd, node, an)

        def slow(carry):
            cs, av, an = carry
            csv = _bc(cs)
            mc = vb == csv
            upd = mc & (key < av)
            av = jnp.where(upd, key, av)
            an = jnp.where(upd, node, an)
            flush(cs, av, an)

            def cond(carry):
                rem, cs, av, an = carry
                return jnp.any(rem)

            def body(carry):
                rem, cs, av, an = carry
                s = jnp.min(jnp.where(rem, vb, IMAX))
                svec = _bc(s)
                segm = vb == svec
                kseg = jnp.where(segm, key, INF)
                is_last = s == vb15

                def mid(args):
                    cs, av, an = args
                    m = jnp.min(kseg)
                    mvec = _bc(m)
                    nodemin = jnp.min(
                        jnp.where(segm & (kseg == mvec), node, NN))
                    wm = lane0 & (mvec < INF)
                    plsc.store_scatter(oval_v, [svec], mvec, mask=wm)
                    plsc.store_scatter(oidx_v, [svec], _bc(nodemin), mask=wm)
                    return cs, av, an

                def last(args):
                    return s, kseg, node

                cs, av, an = lax.cond(is_last, last, mid, (cs, av, an))
                return rem & ~segm, cs, av, an

            rem0 = ~mc
            _, cs, av, an = lax.while_loop(cond, body, (rem0, cs, av, an))
            return cs, av, an

        is_fast = (vb0 == vb15) & (vb0 == cs)
        return lax.cond(is_fast, fast, slow, (cs, av, an))

    cs0 = bidx_v[pl.ds(0, 16)][0]
    carry = (cs0, jnp.full((16,), INF, jnp.float32),
             jnp.zeros((16,), jnp.int32))
    cpa.wait()
    carry = lax.fori_loop(0, NV, step, carry)
    flush(*carry)

    pltpu.sync_copy(oval_v, pval_hbm.at[wid])
    pltpu.sync_copy(oidx_v, pidx_hbm.at[wid])


@functools.partial(
    pl.kernel,
    out_type=(
        jax.ShapeDtypeStruct((OB,), jnp.float32),
        jax.ShapeDtypeStruct((OB * 3,), jnp.float32),
    ),
    mesh=_mesh,
    compiler_params=pltpu.CompilerParams(
        use_tc_tiling_on_sc=True, needs_layout_passes=False),
    scratch_types=[
        pltpu.VMEM((NW, SPW), jnp.float32),
        pltpu.VMEM((NW, SPW), jnp.int32),
        pltpu.VMEM((2, 16, 8, 3), jnp.float32),
        pltpu.VMEM((SPW * 3,), jnp.float32),
        pltpu.VMEM((SPW,), jnp.float32),
        pltpu.SemaphoreType.DMA,
    ],
)
def _phase2(pval_hbm, pidx_hbm, pos_hbm, ptr_hbm, pdir_hbm,
            pv_v, pi_v, posblk_v, pdir_v, ptr_v, semg):
    wid = lax.axis_index("s") * NC + lax.axis_index("c")

    @pl.when(wid < 8)
    def _():
        t = wid
        seg0 = t * SPW
        lanes = _iota16()

        pltpu.sync_copy(pval_hbm.at[:, pl.ds(seg0, SPW)], pv_v)
        pltpu.sync_copy(pidx_hbm.at[:, pl.ds(seg0, SPW)], pi_v)

        def extract(g, pkv, buf):
            rv = pkv - (pkv // 8) * 8
            x = plsc.load_gather(posblk_v, [buf, lanes, rv, _bc(jnp.int32(0))])
            y = plsc.load_gather(posblk_v, [buf, lanes, rv, _bc(jnp.int32(1))])
            z = plsc.load_gather(posblk_v, [buf, lanes, rv, _bc(jnp.int32(2))])
            s = x * x + y * y + z * z
            i = plsc.bitcast(s, jnp.int32)
            i = jnp.int32(0x1FBD1DF5) + (i >> 1)
            r = plsc.bitcast(i, jnp.float32)
            r = 0.5 * (r + s / r)
            r = 0.5 * (r + s / r)
            r = 0.5 * (r + s / r)
            r = jnp.where(s > 0.0, r, 0.0)
            ptr_v[pl.ds(g * 16, 16)] = r
            kv = (_bc(g * 16) + lanes) * 3
            plsc.store_scatter(pdir_v, [kv], x)
            plsc.store_scatter(pdir_v, [kv + 1], y)
            plsc.store_scatter(pdir_v, [kv + 2], z)

        def drain16(i, _):
            pltpu.make_async_copy(
                pos_hbm.at[pl.ds(0, 8), :],
                posblk_v.at[0, 0], semg).wait()
            return 0

        def group(k, pickprev):
            sl = pl.ds(k * 16, 16)
            bv = jnp.full((16,), INF, jnp.float32)
            bi = jnp.full((16,), NN, jnp.int32)
            for w in range(NW):
                av = pv_v[w, sl]
                ai = pi_v[w, sl]
                better = (av < bv) | ((av == bv) & (ai < bi))
                bv = jnp.where(better, av, bv)
                bi = jnp.where(better, ai, bi)
            pickf = jnp.where(bv < INF, bi, 0)
            kmod = k - (k // 2) * 2
            for l in range(16):
                r0 = (pickf[l] // 8) * 8
                pltpu.async_copy(
                    pos_hbm.at[pl.ds(r0, 8), :], posblk_v.at[kmod, l], semg)

            @pl.when(k > 0)
            def _():
                lax.fori_loop(0, 16, drain16, 0)
                extract(k - 1, pickprev, _bc(1 - kmod))

            return pickf

        pickprev = lax.fori_loop(
            0, SPW // 16, group, jnp.zeros((16,), jnp.int32))
        lax.fori_loop(0, 16, drain16, 0)
        extract(SPW // 16 - 1, pickprev, _bc(1))

        pltpu.sync_copy(ptr_v, ptr_hbm.at[pl.ds(seg0, SPW)])
        pltpu.sync_copy(pdir_v, pdir_hbm.at[pl.ds(seg0 * 3, SPW * 3)])


def kernel(x_global_features, h, pos_pxpypz_at_vertex, chi_squared_tracks, batch_idx):
    del x_global_features
    pval, pidx = _phase1(h, chi_squared_tracks, batch_idx.astype(jnp.int32))
    p_tracks, pdir_flat = _phase2(pval, pidx, pos_pxpypz_at_vertex)
    return p_tracks[:NSEG], jnp.reshape(pdir_flat[:NSEG * 3], (NSEG, 3))


# speedup vs baseline: 2.5173x; 1.3681x over previous
import functools

import jax
import jax.numpy as jnp
from jax import lax
from jax.experimental import pallas as pl
from jax.experimental.pallas import tpu as pltpu
from jax.experimental.pallas import tpu_sc as plsc

NN = 100000
NSEG = 1000
OB = 1024
NC, NS = 2, 16
NW = NC * NS
CH = 3200
NV = CH // 16
SPW = 128
INF = float("inf")
IMAX = 2147483647

_mesh = plsc.VectorSubcoreMesh(
    core_axis_name="c", subcore_axis_name="s", num_cores=NC, num_subcores=NS)


def _iota16():
    return lax.broadcasted_iota(jnp.int32, (16,), 0)


def _bc(x):
    return jnp.broadcast_to(x, (16,))


@functools.partial(
    pl.kernel,
    out_type=(
        jax.ShapeDtypeStruct((NW, OB), jnp.float32),
        jax.ShapeDtypeStruct((NW, OB), jnp.int32),
    ),
    mesh=_mesh,
    compiler_params=pltpu.CompilerParams(
        use_tc_tiling_on_sc=False, needs_layout_passes=False),
    scratch_types=[
        pltpu.VMEM((CH, 16), jnp.float32),
        pltpu.VMEM((CH,), jnp.float32),
        pltpu.VMEM((CH,), jnp.int32),
        pltpu.VMEM((OB,), jnp.float32),
        pltpu.VMEM((OB,), jnp.int32),
        pltpu.SemaphoreType.DMA,
    ],
)
def _phase1(h_hbm, chi_hbm, bidx_hbm, pval_hbm, pidx_hbm,
            h16_v, chi_v, bidx_v, oval_v, oidx_v, sema):
    wid = lax.axis_index("s") * NC + lax.axis_index("c")
    base = jnp.minimum(wid * CH, NN - CH)
    lanes = _iota16()
    lane0 = lanes == 0

    cpa = pltpu.async_copy(
        h_hbm.at[pl.ds(base, CH), pl.ds(0, 16)], h16_v, sema)
    pltpu.sync_copy(chi_hbm.at[pl.ds(base, CH)], chi_v)
    pltpu.sync_copy(bidx_hbm.at[pl.ds(base, CH)], bidx_v)

    inf_vec = jnp.full((16,), INF, jnp.float32)
    big_idx = jnp.full((16,), NN, jnp.int32)

    def init(i, _):
        oval_v[pl.ds(i * 16, 16)] = inf_vec
        oidx_v[pl.ds(i * 16, 16)] = big_idx
        return 0

    lax.fori_loop(0, OB // 16, init, 0)

    def flush(cs, av, an):
        m = jnp.min(av)
        mvec = _bc(m)
        nodemin = jnp.min(jnp.where(av == mvec, an, NN))
        csv = _bc(cs)
        wm = lane0 & (mvec < INF)
        plsc.store_scatter(oval_v, [csv], mvec, mask=wm)
        plsc.store_scatter(oidx_v, [csv], _bc(nodemin), mask=wm)

    def step(j, carry):
        cs, av, an = carry
        off = j * 16
        vb = bidx_v[pl.ds(off, 16)]
        vc = chi_v[pl.ds(off, 16)]
        rows = _bc(off) + lanes
        h3 = plsc.load_gather(h16_v, [rows, _bc(jnp.int32(3))])
        h4 = plsc.load_gather(h16_v, [rows, _bc(jnp.int32(4))])
        h5 = plsc.load_gather(h16_v, [rows, _bc(jnp.int32(5))])
        h6 = plsc.load_gather(h16_v, [rows, _bc(jnp.int32(6))])
        filt = (h4 > h3) & (h4 >= h5) & (h4 >= h6)
        key = jnp.where(filt, vc, INF)
        node = _bc(base + off) + lanes
        vb0 = vb[0]
        vb15 = vb[15]

        def fast(carry):
            cs, av, an = carry
            upd = key < av
            return cs, jnp.minimum(av, key), jnp.where(upd, node, an)

        def slow(carry):
            cs, av, an = carry
            csv = _bc(cs)
            mc = vb == csv
            upd = mc & (key < av)
            av = jnp.where(upd, key, av)
            an = jnp.where(upd, node, an)
            flush(cs, av, an)

            def cond(carry):
                rem, cs, av, an = carry
                return jnp.any(rem)

            def body(carry):
                rem, cs, av, an = carry
                s = jnp.min(jnp.where(rem, vb, IMAX))
                svec = _bc(s)
                segm = vb == svec
                kseg = jnp.where(segm, key, INF)
                is_last = s == vb15

                def mid(args):
                    cs, av, an = args
                    m = jnp.min(kseg)
                    mvec = _bc(m)
                    nodemin = jnp.min(
                        jnp.where(segm & (kseg == mvec), node, NN))
                    wm = lane0 & (mvec < INF)
                    plsc.store_scatter(oval_v, [svec], mvec, mask=wm)
                    plsc.store_scatter(oidx_v, [svec], _bc(nodemin), mask=wm)
                    return cs, av, an

                def last(args):
                    return s, kseg, node

                cs, av, an = lax.cond(is_last, last, mid, (cs, av, an))
                return rem & ~segm, cs, av, an

            rem0 = ~mc
            _, cs, av, an = lax.while_loop(cond, body, (rem0, cs, av, an))
            return cs, av, an

        is_fast = (vb0 == vb15) & (vb0 == cs)
        return lax.cond(is_fast, fast, slow, (cs, av, an))

    cs0 = bidx_v[pl.ds(0, 16)][0]
    carry = (cs0, jnp.full((16,), INF, jnp.float32),
             jnp.zeros((16,), jnp.int32))
    cpa.wait()
    carry = lax.fori_loop(0, NV, step, carry)
    flush(*carry)

    pltpu.sync_copy(oval_v, pval_hbm.at[wid])
    pltpu.sync_copy(oidx_v, pidx_hbm.at[wid])


@functools.partial(
    pl.kernel,
    out_type=(
        jax.ShapeDtypeStruct((OB,), jnp.float32),
        jax.ShapeDtypeStruct((OB * 3,), jnp.float32),
    ),
    mesh=_mesh,
    compiler_params=pltpu.CompilerParams(
        use_tc_tiling_on_sc=False, needs_layout_passes=False),
    scratch_types=[
        pltpu.VMEM((NW, SPW), jnp.float32),
        pltpu.VMEM((NW, SPW), jnp.int32),
        pltpu.VMEM((3, SPW), jnp.int32),
        pltpu.VMEM((3, SPW), jnp.float32),
        pltpu.VMEM((SPW * 3,), jnp.float32),
        pltpu.VMEM((SPW,), jnp.float32),
        pltpu.SemaphoreType.DMA,
    ],
)
def _phase2(pval_hbm, pidx_hbm, posf_hbm, ptr_hbm, pdir_hbm,
            pv_v, pi_v, gidx_v, pbuf_v, pdir_v, ptr_v, semg):
    wid = lax.axis_index("s") * NC + lax.axis_index("c")

    @pl.when(wid < 8)
    def _():
        t = wid
        seg0 = t * SPW
        lanes = _iota16()

        pltpu.sync_copy(pval_hbm.at[:, pl.ds(seg0, SPW)], pv_v)
        pltpu.sync_copy(pidx_hbm.at[:, pl.ds(seg0, SPW)], pi_v)

        def combine(k, _):
            sl = pl.ds(k * 16, 16)
            bv = jnp.full((16,), INF, jnp.float32)
            bi = jnp.full((16,), NN, jnp.int32)
            for w in range(NW):
                av = pv_v[w, sl]
                ai = pi_v[w, sl]
                better = (av < bv) | ((av == bv) & (ai < bi))
                bv = jnp.where(better, av, bv)
                bi = jnp.where(better, ai, bi)
            pickf = jnp.where(bv < INF, bi, 0)
            gidx_v[0, sl] = pickf
            gidx_v[1, sl] = pickf + NN
            gidx_v[2, sl] = pickf + 2 * NN
            return 0

        lax.fori_loop(0, SPW // 16, combine, 0)

        cps = [
            pltpu.async_copy(
                posf_hbm.at[gidx_v.at[c]], pbuf_v.at[c], semg)
            for c in range(3)
        ]
        for cp in cps:
            cp.wait()

        def norm_step(k, _):
            sl = pl.ds(k * 16, 16)
            x = pbuf_v[0, sl]
            y = pbuf_v[1, sl]
            z = pbuf_v[2, sl]
            s = x * x + y * y + z * z
            i = plsc.bitcast(s, jnp.int32)
            i = jnp.int32(0x1FBD1DF5) + (i >> 1)
            r = plsc.bitcast(i, jnp.float32)
            r = 0.5 * (r + s / r)
            r = 0.5 * (r + s / r)
            r = 0.5 * (r + s / r)
            r = jnp.where(s > 0.0, r, 0.0)
            ptr_v[sl] = r
            kv = (_bc(k * 16) + lanes) * 3
            plsc.store_scatter(pdir_v, [kv], x)
            plsc.store_scatter(pdir_v, [kv + 1], y)
            plsc.store_scatter(pdir_v, [kv + 2], z)
            return 0

        lax.fori_loop(0, SPW // 16, norm_step, 0)

        pltpu.sync_copy(ptr_v, ptr_hbm.at[pl.ds(seg0, SPW)])
        pltpu.sync_copy(pdir_v, pdir_hbm.at[pl.ds(seg0 * 3, SPW * 3)])


def kernel(x_global_features, h, pos_pxpypz_at_vertex, chi_squared_tracks, batch_idx):
    del x_global_features
    posf = jnp.ravel(pos_pxpypz_at_vertex.T)
    pval, pidx = _phase1(h, chi_squared_tracks, batch_idx.astype(jnp.int32))
    p_tracks, pdir_flat = _phase2(pval, pidx, posf)
    return p_tracks[:NSEG], jnp.reshape(pdir_flat[:NSEG * 3], (NSEG, 3))


# speedup vs baseline: 2.6055x; 1.0350x over previous
import functools

import jax
import jax.numpy as jnp
from jax import lax
from jax.experimental import pallas as pl
from jax.experimental.pallas import tpu as pltpu
from jax.experimental.pallas import tpu_sc as plsc

NN = 100000
NSEG = 1000
OB = 1024
NC, NS = 2, 16
NW = NC * NS
CH = 3200
NV = CH // 16
SPW = 128
INF = float("inf")
IMAX = 2147483647

_mesh = plsc.VectorSubcoreMesh(
    core_axis_name="c", subcore_axis_name="s", num_cores=NC, num_subcores=NS)


def _iota16():
    return lax.broadcasted_iota(jnp.int32, (16,), 0)


def _bc(x):
    return jnp.broadcast_to(x, (16,))


@functools.partial(
    pl.kernel,
    out_type=(
        jax.ShapeDtypeStruct((NW, OB), jnp.float32),
        jax.ShapeDtypeStruct((NW, OB), jnp.int32),
    ),
    mesh=_mesh,
    compiler_params=pltpu.CompilerParams(
        use_tc_tiling_on_sc=False, needs_layout_passes=False),
    scratch_types=[
        pltpu.VMEM((CH, 16), jnp.float32),
        pltpu.VMEM((CH,), jnp.float32),
        pltpu.VMEM((CH,), jnp.int32),
        pltpu.VMEM((CH,), jnp.float32),
        pltpu.VMEM((OB,), jnp.float32),
        pltpu.VMEM((OB,), jnp.int32),
        pltpu.SemaphoreType.DMA,
    ],
)
def _phase1(h_hbm, chi_hbm, bidx_hbm, pval_hbm, pidx_hbm,
            h16_v, chi_v, bidx_v, key_v, oval_v, oidx_v, sema):
    wid = lax.axis_index("s") * NC + lax.axis_index("c")
    base = jnp.minimum(wid * CH, NN - CH)
    lanes = _iota16()
    lane0 = lanes == 0

    cpa = pltpu.async_copy(
        h_hbm.at[pl.ds(base, CH), pl.ds(0, 16)], h16_v, sema)
    pltpu.sync_copy(chi_hbm.at[pl.ds(base, CH)], chi_v)
    pltpu.sync_copy(bidx_hbm.at[pl.ds(base, CH)], bidx_v)

    inf_vec = jnp.full((16,), INF, jnp.float32)
    big_idx = jnp.full((16,), NN, jnp.int32)

    def init(i, _):
        oval_v[pl.ds(i * 16, 16)] = inf_vec
        oidx_v[pl.ds(i * 16, 16)] = big_idx
        return 0

    lax.fori_loop(0, OB // 16, init, 0)

    def flush(cs, av, an):
        m = jnp.min(av)
        mvec = _bc(m)
        nodemin = jnp.min(jnp.where(av == mvec, an, NN))
        csv = _bc(cs)
        wm = lane0 & (mvec < INF)
        plsc.store_scatter(oval_v, [csv], mvec, mask=wm)
        plsc.store_scatter(oidx_v, [csv], _bc(nodemin), mask=wm)

    def step(j, carry):
        cs, av, an = carry
        off = j * 16
        vb = bidx_v[pl.ds(off, 16)]
        key = key_v[pl.ds(off, 16)]
        node = _bc(base + off) + lanes
        vb0 = vb[0]
        vb15 = vb[15]

        def fast(carry):
            cs, av, an = carry
            upd = key < av
            return cs, jnp.minimum(av, key), jnp.where(upd, node, an)

        def slow(carry):
            cs, av, an = carry
            csv = _bc(cs)
            mc = vb == csv
            upd = mc & (key < av)
            av = jnp.where(upd, key, av)
            an = jnp.where(upd, node, an)
            flush(cs, av, an)

            def cond(carry):
                rem, cs, av, an = carry
                return jnp.any(rem)

            def body(carry):
                rem, cs, av, an = carry
                s = jnp.min(jnp.where(rem, vb, IMAX))
                svec = _bc(s)
                segm = vb == svec
                kseg = jnp.where(segm, key, INF)
                is_last = s == vb15

                def mid(args):
                    cs, av, an = args
                    m = jnp.min(kseg)
                    mvec = _bc(m)
                    nodemin = jnp.min(
                        jnp.where(segm & (kseg == mvec), node, NN))
                    wm = lane0 & (mvec < INF)
                    plsc.store_scatter(oval_v, [svec], mvec, mask=wm)
                    plsc.store_scatter(oidx_v, [svec], _bc(nodemin), mask=wm)
                    return cs, av, an

                def last(args):
                    return s, kseg, node

                cs, av, an = lax.cond(is_last, last, mid, (cs, av, an))
                return rem & ~segm, cs, av, an

            rem0 = ~mc
            _, cs, av, an = lax.while_loop(cond, body, (rem0, cs, av, an))
            return cs, av, an

        is_fast = (vb0 == vb15) & (vb0 == cs)
        return lax.cond(is_fast, fast, slow, (cs, av, an))

    cs0 = bidx_v[pl.ds(0, 16)][0]
    carry = (cs0, jnp.full((16,), INF, jnp.float32),
             jnp.zeros((16,), jnp.int32))
    cpa.wait()

    @functools.partial(plsc.parallel_loop, 0, NV, unroll=4)
    def _(j):
        off = j * 16
        vc = chi_v[pl.ds(off, 16)]
        rows = _bc(off) + lanes
        h3 = plsc.load_gather(h16_v, [rows, _bc(jnp.int32(3))])
        h4 = plsc.load_gather(h16_v, [rows, _bc(jnp.int32(4))])
        h5 = plsc.load_gather(h16_v, [rows, _bc(jnp.int32(5))])
        h6 = plsc.load_gather(h16_v, [rows, _bc(jnp.int32(6))])
        filt = (h4 > h3) & (h4 >= h5) & (h4 >= h6)
        key_v[pl.ds(off, 16)] = jnp.where(filt, vc, INF)

    carry = lax.fori_loop(0, NV, step, carry)
    flush(*carry)

    pltpu.sync_copy(oval_v, pval_hbm.at[wid])
    pltpu.sync_copy(oidx_v, pidx_hbm.at[wid])


@functools.partial(
    pl.kernel,
    out_type=(
        jax.ShapeDtypeStruct((OB,), jnp.float32),
        jax.ShapeDtypeStruct((OB * 3,), jnp.float32),
    ),
    mesh=_mesh,
    compiler_params=pltpu.CompilerParams(
        use_tc_tiling_on_sc=False, needs_layout_passes=False),
    scratch_types=[
        pltpu.VMEM((NW, SPW), jnp.float32),
        pltpu.VMEM((NW, SPW), jnp.int32),
        pltpu.VMEM((3, SPW), jnp.int32),
        pltpu.VMEM((3, SPW), jnp.float32),
        pltpu.VMEM((SPW * 3,), jnp.float32),
        pltpu.VMEM((SPW,), jnp.float32),
        pltpu.SemaphoreType.DMA,
    ],
)
def _phase2(pval_hbm, pidx_hbm, posf_hbm, ptr_hbm, pdir_hbm,
            pv_v, pi_v, gidx_v, pbuf_v, pdir_v, ptr_v, semg):
    wid = lax.axis_index("s") * NC + lax.axis_index("c")

    @pl.when(wid < 8)
    def _():
        t = wid
        seg0 = t * SPW
        lanes = _iota16()

        pltpu.sync_copy(pval_hbm.at[:, pl.ds(seg0, SPW)], pv_v)
        pltpu.sync_copy(pidx_hbm.at[:, pl.ds(seg0, SPW)], pi_v)

        def combine(k, _):
            sl = pl.ds(k * 16, 16)
            bv = jnp.full((16,), INF, jnp.float32)
            bi = jnp.full((16,), NN, jnp.int32)
            for w in range(NW):
                av = pv_v[w, sl]
                ai = pi_v[w, sl]
                better = (av < bv) | ((av == bv) & (ai < bi))
                bv = jnp.where(better, av, bv)
                bi = jnp.where(better, ai, bi)
            pickf = jnp.where(bv < INF, bi, 0)
            gidx_v[0, sl] = pickf
            gidx_v[1, sl] = pickf + NN
            gidx_v[2, sl] = pickf + 2 * NN
            return 0

        lax.fori_loop(0, SPW // 16, combine, 0)

        cps = [
            pltpu.async_copy(
                posf_hbm.at[gidx_v.at[c]], pbuf_v.at[c], semg)
            for c in range(3)
        ]
        for cp in cps:
            cp.wait()

        def norm_step(k, _):
            sl = pl.ds(k * 16, 16)
            x = pbuf_v[0, sl]
            y = pbuf_v[1, sl]
            z = pbuf_v[2, sl]
            s = x * x + y * y + z * z
            i = plsc.bitcast(s, jnp.int32)
            i = jnp.int32(0x1FBD1DF5) + (i >> 1)
            r = plsc.bitcast(i, jnp.float32)
            r = 0.5 * (r + s / r)
            r = 0.5 * (r + s / r)
            r = 0.5 * (r + s / r)
            r = jnp.where(s > 0.0, r, 0.0)
            ptr_v[sl] = r
            kv = (_bc(k * 16) + lanes) * 3
            plsc.store_scatter(pdir_v, [kv], x)
            plsc.store_scatter(pdir_v, [kv + 1], y)
            plsc.store_scatter(pdir_v, [kv + 2], z)
            return 0

        lax.fori_loop(0, SPW // 16, norm_step, 0)

        pltpu.sync_copy(ptr_v, ptr_hbm.at[pl.ds(seg0, SPW)])
        pltpu.sync_copy(pdir_v, pdir_hbm.at[pl.ds(seg0 * 3, SPW * 3)])


def kernel(x_global_features, h, pos_pxpypz_at_vertex, chi_squared_tracks, batch_idx):
    del x_global_features
    posf = jnp.ravel(pos_pxpypz_at_vertex.T)
    pval, pidx = _phase1(h, chi_squared_tracks, batch_idx.astype(jnp.int32))
    p_tracks, pdir_flat = _phase2(pval, pidx, posf)
    return p_tracks[:NSEG], jnp.reshape(pdir_flat[:NSEG * 3], (NSEG, 3))


# speedup vs baseline: 2.6171x; 1.0044x over previous
import functools

import jax
import jax.numpy as jnp
from jax import lax
from jax.experimental import pallas as pl
from jax.experimental.pallas import tpu as pltpu
from jax.experimental.pallas import tpu_sc as plsc

NN = 100000
NSEG = 1000
OB = 1024
NC, NS = 2, 16
NW = NC * NS
CH = 3200
NV = CH // 16
SPW = 128
INF = float("inf")
IMAX = 2147483647

_mesh = plsc.VectorSubcoreMesh(
    core_axis_name="c", subcore_axis_name="s", num_cores=NC, num_subcores=NS)


def _iota16():
    return lax.broadcasted_iota(jnp.int32, (16,), 0)


def _bc(x):
    return jnp.broadcast_to(x, (16,))


@functools.partial(
    pl.kernel,
    out_type=(
        jax.ShapeDtypeStruct((NW, OB), jnp.float32),
        jax.ShapeDtypeStruct((NW, OB), jnp.int32),
    ),
    mesh=_mesh,
    compiler_params=pltpu.CompilerParams(
        use_tc_tiling_on_sc=False, needs_layout_passes=False),
    scratch_types=[
        pltpu.VMEM((CH, 8), jnp.float32),
        pltpu.VMEM((CH,), jnp.float32),
        pltpu.VMEM((CH,), jnp.int32),
        pltpu.VMEM((OB,), jnp.float32),
        pltpu.VMEM((OB,), jnp.int32),
        pltpu.SemaphoreType.DMA,
        pltpu.SemaphoreType.DMA,
    ],
)
def _phase1(h_hbm, chi_hbm, bidx_hbm, pval_hbm, pidx_hbm,
            h16_v, chi_v, bidx_v, oval_v, oidx_v, sema, semb):
    wid = lax.axis_index("s") * NC + lax.axis_index("c")
    base = jnp.minimum(wid * CH, NN - CH)
    lanes = _iota16()
    lane0 = lanes == 0

    cpa = pltpu.async_copy(
        h_hbm.at[pl.ds(base, CH), pl.ds(0, 8)], h16_v, sema)
    cpc = pltpu.async_copy(chi_hbm.at[pl.ds(base, CH)], chi_v, semb)
    cpb = pltpu.async_copy(bidx_hbm.at[pl.ds(base, CH)], bidx_v, semb)

    inf_vec = jnp.full((16,), INF, jnp.float32)
    big_idx = jnp.full((16,), NN, jnp.int32)

    def init(i, _):
        oval_v[pl.ds(i * 16, 16)] = inf_vec
        oidx_v[pl.ds(i * 16, 16)] = big_idx
        return 0

    lax.fori_loop(0, OB // 16, init, 0)

    def flush(cs, av, an):
        m = jnp.min(av)
        mvec = _bc(m)
        nodemin = jnp.min(jnp.where(av == mvec, an, NN))
        csv = _bc(cs)
        wm = lane0 & (mvec < INF)
        plsc.store_scatter(oval_v, [csv], mvec, mask=wm)
        plsc.store_scatter(oidx_v, [csv], _bc(nodemin), mask=wm)

    def step(j, carry):
        cs, av, an = carry
        off = j * 16
        vb = bidx_v[pl.ds(off, 16)]
        vc = chi_v[pl.ds(off, 16)]
        rows = _bc(off) + lanes
        h3 = plsc.load_gather(h16_v, [rows, _bc(jnp.int32(3))])
        h4 = plsc.load_gather(h16_v, [rows, _bc(jnp.int32(4))])
        h5 = plsc.load_gather(h16_v, [rows, _bc(jnp.int32(5))])
        h6 = plsc.load_gather(h16_v, [rows, _bc(jnp.int32(6))])
        filt = (h4 > h3) & (h4 >= h5) & (h4 >= h6)
        key = jnp.where(filt, vc, INF)
        node = _bc(base + off) + lanes
        vb0 = vb[0]
        vb15 = vb[15]

        def fast(carry):
            cs, av, an = carry
            upd = key < av
            return cs, jnp.minimum(av, key), jnp.where(upd, node, an)

        def slow(carry):
            cs, av, an = carry
            csv = _bc(cs)
            mc = vb == csv
            upd = mc & (key < av)
            av = jnp.where(upd, key, av)
            an = jnp.where(upd, node, an)
            flush(cs, av, an)

            def cond(carry):
                rem, cs, av, an = carry
                return jnp.any(rem)

            def body(carry):
                rem, cs, av, an = carry
                s = jnp.min(jnp.where(rem, vb, IMAX))
                svec = _bc(s)
                segm = vb == svec
                kseg = jnp.where(segm, key, INF)
                is_last = s == vb15

                def mid(args):
                    cs, av, an = args
                    m = jnp.min(kseg)
                    mvec = _bc(m)
                    nodemin = jnp.min(
                        jnp.where(segm & (kseg == mvec), node, NN))
                    wm = lane0 & (mvec < INF)
                    plsc.store_scatter(oval_v, [svec], mvec, mask=wm)
                    plsc.store_scatter(oidx_v, [svec], _bc(nodemin), mask=wm)
                    return cs, av, an

                def last(args):
                    return s, kseg, node

                cs, av, an = lax.cond(is_last, last, mid, (cs, av, an))
                return rem & ~segm, cs, av, an

            rem0 = ~mc
            _, cs, av, an = lax.while_loop(cond, body, (rem0, cs, av, an))
            return cs, av, an

        is_fast = (vb0 == vb15) & (vb0 == cs)
        return lax.cond(is_fast, fast, slow, (cs, av, an))

    cpc.wait()
    cpb.wait()
    cs0 = bidx_v[pl.ds(0, 16)][0]
    carry = (cs0, jnp.full((16,), INF, jnp.float32),
             jnp.zeros((16,), jnp.int32))
    cpa.wait()

    def step2(jj, carry):
        return step(jj * 2 + 1, step(jj * 2, carry))

    carry = lax.fori_loop(0, NV // 2, step2, carry)
    flush(*carry)

    pltpu.sync_copy(oval_v, pval_hbm.at[wid])
    pltpu.sync_copy(oidx_v, pidx_hbm.at[wid])


@functools.partial(
    pl.kernel,
    out_type=(
        jax.ShapeDtypeStruct((OB,), jnp.float32),
        jax.ShapeDtypeStruct((OB * 3,), jnp.float32),
    ),
    mesh=_mesh,
    compiler_params=pltpu.CompilerParams(
        use_tc_tiling_on_sc=False, needs_layout_passes=False),
    scratch_types=[
        pltpu.VMEM((NW, SPW), jnp.float32),
        pltpu.VMEM((NW, SPW), jnp.int32),
        pltpu.VMEM((3, SPW), jnp.int32),
        pltpu.VMEM((3, SPW), jnp.float32),
        pltpu.VMEM((SPW * 3,), jnp.float32),
        pltpu.VMEM((SPW,), jnp.float32),
        pltpu.SemaphoreType.DMA,
    ],
)
def _phase2(pval_hbm, pidx_hbm, posf_hbm, ptr_hbm, pdir_hbm,
            pv_v, pi_v, gidx_v, pbuf_v, pdir_v, ptr_v, semg):
    wid = lax.axis_index("s") * NC + lax.axis_index("c")

    @pl.when(wid < 8)
    def _():
        t = wid
        seg0 = t * SPW
        lanes = _iota16()

        pltpu.sync_copy(pval_hbm.at[:, pl.ds(seg0, SPW)], pv_v)
        pltpu.sync_copy(pidx_hbm.at[:, pl.ds(seg0, SPW)], pi_v)

        def combine(k, _):
            sl = pl.ds(k * 16, 16)
            bv = jnp.full((16,), INF, jnp.float32)
            bi = jnp.full((16,), NN, jnp.int32)
            for w in range(NW):
                av = pv_v[w, sl]
                ai = pi_v[w, sl]
                better = (av < bv) | ((av == bv) & (ai < bi))
                bv = jnp.where(better, av, bv)
                bi = jnp.where(better, ai, bi)
            pickf = jnp.where(bv < INF, bi, 0)
            gidx_v[0, sl] = pickf
            gidx_v[1, sl] = pickf + NN
            gidx_v[2, sl] = pickf + 2 * NN
            return 0

        lax.fori_loop(0, SPW // 16, combine, 0)

        cps = [
            pltpu.async_copy(
                posf_hbm.at[gidx_v.at[c]], pbuf_v.at[c], semg)
            for c in range(3)
        ]
        for cp in cps:
            cp.wait()

        def norm_step(k, _):
            sl = pl.ds(k * 16, 16)
            x = pbuf_v[0, sl]
            y = pbuf_v[1, sl]
            z = pbuf_v[2, sl]
            s = x * x + y * y + z * z
            i = plsc.bitcast(s, jnp.int32)
            i = jnp.int32(0x1FBD1DF5) + (i >> 1)
            r = plsc.bitcast(i, jnp.float32)
            r = 0.5 * (r + s / r)
            r = 0.5 * (r + s / r)
            r = 0.5 * (r + s / r)
            r = jnp.where(s > 0.0, r, 0.0)
            ptr_v[sl] = r
            kv = (_bc(k * 16) + lanes) * 3
            plsc.store_scatter(pdir_v, [kv], x)
            plsc.store_scatter(pdir_v, [kv + 1], y)
            plsc.store_scatter(pdir_v, [kv + 2], z)
            return 0

        lax.fori_loop(0, SPW // 16, norm_step, 0)

        pltpu.sync_copy(ptr_v, ptr_hbm.at[pl.ds(seg0, SPW)])
        pltpu.sync_copy(pdir_v, pdir_hbm.at[pl.ds(seg0 * 3, SPW * 3)])


def kernel(x_global_features, h, pos_pxpypz_at_vertex, chi_squared_tracks, batch_idx):
    del x_global_features
    posf = jnp.ravel(pos_pxpypz_at_vertex.T)
    pval, pidx = _phase1(h, chi_squared_tracks, batch_idx.astype(jnp.int32))
    p_tracks, pdir_flat = _phase2(pval, pidx, posf)
    return p_tracks[:NSEG], jnp.reshape(pdir_flat[:NSEG * 3], (NSEG, 3))
